# Initial kernel scaffold; baseline (speedup 1.0000x reference)
import jax
import jax.numpy as jnp
from jax.experimental import pallas as pl


def kernel(hidden_states, router_logits, W_gate, W_up, W_down):
    raise NotImplementedError("write your pallas kernel here")


# speedup vs baseline: 2.3252x; 2.3252x over previous
import functools

import jax
import jax.numpy as jnp
from jax.experimental import pallas as pl
from jax.experimental.pallas import tpu as pltpu

NUM_EXPERTS = 64
TOP_K = 2
HIDDEN = 1024
D_FF = 512
TOKENS = 128


def _moe_kernel(x_ref, logits_ref, wg_ref, wu_ref, wd_ref, out_ref,
                w1_ref, w2_ref, a1_ref, a2_ref):
    e = pl.program_id(0)

    @pl.when(e == 0)
    def _routing():
        logits = logits_ref[...]
        m1 = jnp.max(logits, axis=1, keepdims=True)
        a1 = jnp.argmax(logits, axis=1).reshape(TOKENS, 1)
        cols = jax.lax.broadcasted_iota(jnp.int32, (TOKENS, NUM_EXPERTS), 1)
        masked = jnp.where(cols == a1, -jnp.inf, logits)
        m2 = jnp.max(masked, axis=1, keepdims=True)
        a2 = jnp.argmax(masked, axis=1).reshape(TOKENS, 1)
        e2 = jnp.exp(m2 - m1)
        w1_ref[...] = 1.0 / (1.0 + e2)
        w2_ref[...] = e2 / (1.0 + e2)
        a1_ref[...] = a1
        a2_ref[...] = a2
        out_ref[...] = jnp.zeros_like(out_ref)

    x = x_ref[...]
    gate = jnp.dot(x, wg_ref[0], preferred_element_type=jnp.float32)
    up = jnp.dot(x, wu_ref[0], preferred_element_type=jnp.float32)
    act = gate * jax.nn.sigmoid(gate) * up
    y = jnp.dot(act, wd_ref[0], preferred_element_type=jnp.float32)

    w = (jnp.where(a1_ref[...] == e, w1_ref[...], 0.0)
         + jnp.where(a2_ref[...] == e, w2_ref[...], 0.0))
    out_ref[...] += y * w


@jax.jit
def kernel(hidden_states, router_logits, W_gate, W_up, W_down):
    return pl.pallas_call(
        _moe_kernel,
        grid=(NUM_EXPERTS,),
        in_specs=[
            pl.BlockSpec((TOKENS, HIDDEN), lambda e: (0, 0)),
            pl.BlockSpec((TOKENS, NUM_EXPERTS), lambda e: (0, 0)),
            pl.BlockSpec((1, HIDDEN, D_FF), lambda e: (e, 0, 0)),
            pl.BlockSpec((1, HIDDEN, D_FF), lambda e: (e, 0, 0)),
            pl.BlockSpec((1, D_FF, HIDDEN), lambda e: (e, 0, 0)),
        ],
        out_specs=pl.BlockSpec((TOKENS, HIDDEN), lambda e: (0, 0)),
        out_shape=jax.ShapeDtypeStruct((TOKENS, HIDDEN), jnp.float32),
        scratch_shapes=[
            pltpu.VMEM((TOKENS, 1), jnp.float32),
            pltpu.VMEM((TOKENS, 1), jnp.float32),
            pltpu.VMEM((TOKENS, 1), jnp.int32),
            pltpu.VMEM((TOKENS, 1), jnp.int32),
        ],
        compiler_params=pltpu.CompilerParams(
            dimension_semantics=("arbitrary",),
        ),
    )(hidden_states, router_logits, W_gate, W_up, W_down)
